# baseline (speedup 1.0000x reference)
import functools

import jax
import jax.numpy as jnp
from jax import lax
from jax.experimental import pallas as pl
from jax.experimental.pallas import tpu as pltpu
from jax.experimental.pallas import tpu_sc as plsc

B = 16384
F = 26
V = 100000
D = 50
GW = 200
DP = 64
NUM = 13
LT = F * DP // 128
EDP = F * DP

NC, NS = 2, 16
NW = NC * NS
BF = B * F
ROWS_PER_W = BF // NW
CHUNK = 128
NCHUNK = ROWS_PER_W // CHUNK
NG = CHUNK // 16

_sc_mesh = plsc.VectorSubcoreMesh(
    core_axis_name="c", subcore_axis_name="s", num_cores=NC, num_subcores=NS
)


def _extract(buf, shf_v, outb, c):
    iota = lax.iota(jnp.int32, 16)
    iota2 = iota // 2
    colb = (iota % 2) * DP

    def group(g, carry):
        rows = g * 16 + iota
        orows = g * 8 + iota2
        shift = plsc.load_gather(shf_v, [jnp.zeros((16,), jnp.int32) + c, rows])

        def col(j, carry2):
            vals = plsc.load_gather(buf, [rows, jnp.minimum(shift + j, GW - 1)])
            plsc.store_scatter(outb, [orows, colb + j], vals)
            return carry2

        lax.fori_loop(0, DP, col, 0, unroll=8)
        return carry

    lax.fori_loop(0, NG, group, 0)


@functools.partial(
    pl.kernel,
    out_type=jax.ShapeDtypeStruct((BF // 2, 128), jnp.float32),
    mesh=_sc_mesh,
    compiler_params=pltpu.CompilerParams(
        use_tc_tiling_on_sc=False, needs_layout_passes=False
    ),
    scratch_types=[
        pltpu.VMEM((NCHUNK, CHUNK), jnp.int32),
        pltpu.VMEM((NCHUNK, CHUNK), jnp.int32),
        pltpu.VMEM((CHUNK, GW), jnp.float32),
        pltpu.VMEM((CHUNK, GW), jnp.float32),
        pltpu.VMEM((CHUNK // 2, 128), jnp.float32),
        pltpu.VMEM((CHUNK // 2, 128), jnp.float32),
        pltpu.SemaphoreType.DMA,
        pltpu.SemaphoreType.DMA,
    ],
)
def _sc_gather(tab_hbm, gidx_hbm, shf_hbm, out_hbm, gidx_v, shf_v,
               buf_a, buf_b, outb_a, outb_b, sem_a, sem_b):
    wid = lax.axis_index("s") * NC + lax.axis_index("c")
    base = wid * ROWS_PER_W
    pltpu.sync_copy(gidx_hbm.at[wid], gidx_v)
    pltpu.sync_copy(shf_hbm.at[wid], shf_v)
    pltpu.async_copy(tab_hbm.at[gidx_v.at[0]], buf_a, sem_a)

    def body(i, carry):
        c0 = i * 2
        c1 = c0 + 1
        pltpu.async_copy(tab_hbm.at[gidx_v.at[c1]], buf_b, sem_b)
        pltpu.make_async_copy(tab_hbm.at[gidx_v.at[c0]], buf_a, sem_a).wait()
        _extract(buf_a, shf_v, outb_a, c0)
        pltpu.sync_copy(
            outb_a, out_hbm.at[pl.ds(base // 2 + c0 * (CHUNK // 2), CHUNK // 2)]
        )

        @pl.when(c1 + 1 < NCHUNK)
        def _():
            pltpu.async_copy(tab_hbm.at[gidx_v.at[c1 + 1]], buf_a, sem_a)

        pltpu.make_async_copy(tab_hbm.at[gidx_v.at[c1]], buf_b, sem_b).wait()
        _extract(buf_b, shf_v, outb_b, c1)
        pltpu.sync_copy(
            outb_b, out_hbm.at[pl.ds(base // 2 + c1 * (CHUNK // 2), CHUNK // 2)]
        )
        return carry

    lax.fori_loop(0, NCHUNK // 2, body, 0)


BB = 1024


def _mlp_body(emb4, xnum, w1t, w1n, b1, w2, b2, w3, b3, w4, b4, out):
    f32 = jnp.float32
    hi = lax.Precision.HIGHEST
    x = jnp.concatenate(
        [emb4[:, l].reshape(BB, 128) for l in range(LT)], axis=1
    )
    h = jnp.dot(x, w1t[...], preferred_element_type=f32, precision=hi)
    h += jnp.dot(xnum[...], w1n[...], preferred_element_type=f32, precision=hi)
    h = jnp.maximum(h + b1[...], 0.0)
    h = jnp.maximum(jnp.dot(h, w2[...], preferred_element_type=f32, precision=hi) + b2[...], 0.0)
    h = jnp.maximum(jnp.dot(h, w3[...], preferred_element_type=f32, precision=hi) + b3[...], 0.0)
    out[...] = jnp.dot(h, w4[...], preferred_element_type=f32, precision=hi) + b4[...]


def _mlp(emb4, xnum, w1t, w1n, b1, w2, b2, w3, b3, w4, b4):
    grid = (B // BB,)
    full = lambda s: pl.BlockSpec(s, lambda i: (0,) * len(s))
    return pl.pallas_call(
        _mlp_body,
        grid=grid,
        in_specs=[
            pl.BlockSpec((BB // 8, LT, 8, 128), lambda i: (i, 0, 0, 0)),
            pl.BlockSpec((BB, NUM), lambda i: (i, 0)),
            full((EDP, 512)),
            full((NUM, 512)),
            full((1, 512)),
            full((512, 256)),
            full((1, 256)),
            full((256, 32)),
            full((1, 32)),
            full((32, 1)),
            full((1, 1)),
        ],
        out_specs=pl.BlockSpec((BB, 1), lambda i: (i, 0)),
        out_shape=jax.ShapeDtypeStruct((B, 1), jnp.float32),
    )(emb4, xnum, w1t, w1n, b1, w2, b2, w3, b3, w4, b4)


def kernel(x_num, x_cat, tables, W1, b1, W2, b2, W3, b3, W4, b4):
    idx = x_cat + (jnp.arange(F, dtype=jnp.int32) * V)[None, :]
    idxp = idx.reshape(B // 8, 8, F // 2, 2).transpose(0, 2, 1, 3)
    idxp = idxp.reshape(NW, NCHUNK, CHUNK)
    gidx = idxp // 4
    shf = (idxp % 4) * D
    tab = tables.reshape(F * V // 4, GW)
    out = _sc_gather(tab, gidx, shf)
    emb4 = out.reshape(B // 8, LT, 8, 128)
    w1e = W1[: F * D].reshape(F, D, 512)
    w1t = jnp.concatenate(
        [w1e, jnp.zeros((F, DP - D, 512), jnp.float32)], axis=1
    ).reshape(EDP, 512)
    return _mlp(
        emb4,
        x_num,
        w1t,
        W1[F * D :],
        b1.reshape(1, -1),
        W2,
        b2.reshape(1, -1),
        W3,
        b3.reshape(1, -1),
        W4,
        b4.reshape(1, -1),
    )


# speedup vs baseline: 1.0153x; 1.0153x over previous
import functools

import jax
import jax.numpy as jnp
from jax import lax
from jax.experimental import pallas as pl
from jax.experimental.pallas import tpu as pltpu
from jax.experimental.pallas import tpu_sc as plsc

B = 16384
F = 26
V = 100000
D = 50
GW = 200
DP = 64
NUM = 13
LT = F * DP // 128
EDP = F * DP

NC, NS = 2, 16
NW = NC * NS
BF = B * F
ROWS_PER_W = BF // NW
CHUNK = 128
NCHUNK = ROWS_PER_W // CHUNK
NG = CHUNK // 16

_sc_mesh = plsc.VectorSubcoreMesh(
    core_axis_name="c", subcore_axis_name="s", num_cores=NC, num_subcores=NS
)


def _extract(buf, shf_v, outb, c):
    iota = lax.iota(jnp.int32, 16)
    iota2 = iota // 2
    colb = (iota % 2) * DP

    def group(g, carry):
        rows = g * 16 + iota
        orows = g * 8 + iota2
        shift = plsc.load_gather(shf_v, [jnp.zeros((16,), jnp.int32) + c, rows])

        def col(j, carry2):
            vals = plsc.load_gather(buf, [rows, shift + j])
            plsc.store_scatter(outb, [orows, colb + j], vals)
            return carry2

        lax.fori_loop(0, D, col, 0, unroll=8)

        def pad(j, carry2):
            plsc.store_scatter(outb, [orows, colb + j], jnp.zeros((16,), jnp.float32))
            return carry2

        lax.fori_loop(D, DP, pad, 0, unroll=7)
        return carry

    lax.fori_loop(0, NG, group, 0)


@functools.partial(
    pl.kernel,
    out_type=jax.ShapeDtypeStruct((BF // 2, 128), jnp.float32),
    mesh=_sc_mesh,
    compiler_params=pltpu.CompilerParams(
        use_tc_tiling_on_sc=False, needs_layout_passes=False
    ),
    scratch_types=[
        pltpu.VMEM((NCHUNK, CHUNK), jnp.int32),
        pltpu.VMEM((NCHUNK, CHUNK), jnp.int32),
        pltpu.VMEM((CHUNK, GW), jnp.float32),
        pltpu.VMEM((CHUNK, GW), jnp.float32),
        pltpu.VMEM((CHUNK // 2, 128), jnp.float32),
        pltpu.VMEM((CHUNK // 2, 128), jnp.float32),
        pltpu.SemaphoreType.DMA,
        pltpu.SemaphoreType.DMA,
    ],
)
def _sc_gather(tab_hbm, gidx_hbm, shf_hbm, out_hbm, gidx_v, shf_v,
               buf_a, buf_b, outb_a, outb_b, sem_a, sem_b):
    wid = lax.axis_index("s") * NC + lax.axis_index("c")
    base = wid * ROWS_PER_W
    pltpu.sync_copy(gidx_hbm.at[wid], gidx_v)
    pltpu.sync_copy(shf_hbm.at[wid], shf_v)
    pltpu.async_copy(tab_hbm.at[gidx_v.at[0]], buf_a, sem_a)

    def body(i, carry):
        c0 = i * 2
        c1 = c0 + 1
        pltpu.async_copy(tab_hbm.at[gidx_v.at[c1]], buf_b, sem_b)
        pltpu.make_async_copy(tab_hbm.at[gidx_v.at[c0]], buf_a, sem_a).wait()
        _extract(buf_a, shf_v, outb_a, c0)
        pltpu.sync_copy(
            outb_a, out_hbm.at[pl.ds(base // 2 + c0 * (CHUNK // 2), CHUNK // 2)]
        )

        @pl.when(c1 + 1 < NCHUNK)
        def _():
            pltpu.async_copy(tab_hbm.at[gidx_v.at[c1 + 1]], buf_a, sem_a)

        pltpu.make_async_copy(tab_hbm.at[gidx_v.at[c1]], buf_b, sem_b).wait()
        _extract(buf_b, shf_v, outb_b, c1)
        pltpu.sync_copy(
            outb_b, out_hbm.at[pl.ds(base // 2 + c1 * (CHUNK // 2), CHUNK // 2)]
        )
        return carry

    lax.fori_loop(0, NCHUNK // 2, body, 0)


BB = 1024


def _mlp_body(emb4, xnum, w1t, w1n, b1, w2, b2, w3, b3, w4, b4, out):
    f32 = jnp.float32
    hi = lax.Precision.HIGHEST
    x = jnp.concatenate(
        [emb4[:, l].reshape(BB, 128) for l in range(LT)], axis=1
    )
    h = jnp.dot(x, w1t[...], preferred_element_type=f32, precision=hi)
    h += jnp.dot(xnum[...], w1n[...], preferred_element_type=f32, precision=hi)
    h = jnp.maximum(h + b1[...], 0.0)
    h = jnp.maximum(jnp.dot(h, w2[...], preferred_element_type=f32, precision=hi) + b2[...], 0.0)
    h = jnp.maximum(jnp.dot(h, w3[...], preferred_element_type=f32, precision=hi) + b3[...], 0.0)
    out[...] = jnp.dot(h, w4[...], preferred_element_type=f32, precision=hi) + b4[...]


def _mlp(emb4, xnum, w1t, w1n, b1, w2, b2, w3, b3, w4, b4):
    grid = (B // BB,)
    full = lambda s: pl.BlockSpec(s, lambda i: (0,) * len(s))
    return pl.pallas_call(
        _mlp_body,
        grid=grid,
        in_specs=[
            pl.BlockSpec((BB // 8, LT, 8, 128), lambda i: (i, 0, 0, 0)),
            pl.BlockSpec((BB, NUM), lambda i: (i, 0)),
            full((EDP, 512)),
            full((NUM, 512)),
            full((1, 512)),
            full((512, 256)),
            full((1, 256)),
            full((256, 32)),
            full((1, 32)),
            full((32, 1)),
            full((1, 1)),
        ],
        out_specs=pl.BlockSpec((BB, 1), lambda i: (i, 0)),
        out_shape=jax.ShapeDtypeStruct((B, 1), jnp.float32),
    )(emb4, xnum, w1t, w1n, b1, w2, b2, w3, b3, w4, b4)


def kernel(x_num, x_cat, tables, W1, b1, W2, b2, W3, b3, W4, b4):
    idx = x_cat + (jnp.arange(F, dtype=jnp.int32) * V)[None, :]
    idxp = idx.reshape(B // 8, 8, F // 2, 2).transpose(0, 2, 1, 3)
    idxp = idxp.reshape(NW, NCHUNK, CHUNK)
    gidx = idxp // 4
    shf = (idxp % 4) * D
    tab = tables.reshape(F * V // 4, GW)
    out = _sc_gather(tab, gidx, shf)
    emb4 = out.reshape(B // 8, LT, 8, 128)
    w1e = W1[: F * D].reshape(F, D, 512)
    w1t = jnp.concatenate(
        [w1e, jnp.zeros((F, DP - D, 512), jnp.float32)], axis=1
    ).reshape(EDP, 512)
    return _mlp(
        emb4,
        x_num,
        w1t,
        W1[F * D :],
        b1.reshape(1, -1),
        W2,
        b2.reshape(1, -1),
        W3,
        b3.reshape(1, -1),
        W4,
        b4.reshape(1, -1),
    )
